# baseline (speedup 1.0000x reference)
import jax
import jax.numpy as jnp
from jax.experimental import pallas as pl
from jax.experimental.pallas import tpu as pltpu

TOKENS = 8192
D_MODEL = 4096
UNITS = 64
TEMP = 0.5

BLOCK_M = 256


def _router_kernel(x_ref, w_ref, b_ref, o_ref):
    logits = jnp.dot(x_ref[...], w_ref[...], preferred_element_type=jnp.float32)
    logits = logits + b_ref[...]
    o_ref[...] = jax.nn.sigmoid(logits / (TEMP + 1e-08))


def kernel(inputs, proj, logit_bias):
    bias2d = logit_bias.reshape(1, UNITS)
    grid = (TOKENS // BLOCK_M,)
    return pl.pallas_call(
        _router_kernel,
        grid=grid,
        in_specs=[
            pl.BlockSpec((BLOCK_M, D_MODEL), lambda i: (i, 0)),
            pl.BlockSpec((D_MODEL, UNITS), lambda i: (0, 0)),
            pl.BlockSpec((1, UNITS), lambda i: (0, 0)),
        ],
        out_specs=pl.BlockSpec((BLOCK_M, UNITS), lambda i: (i, 0)),
        out_shape=jax.ShapeDtypeStruct((TOKENS, UNITS), jnp.float32),
        compiler_params=pltpu.CompilerParams(
            dimension_semantics=("arbitrary",),
        ),
    )(inputs, proj, bias2d)


# speedup vs baseline: 1.1371x; 1.1371x over previous
import jax
import jax.numpy as jnp
from jax.experimental import pallas as pl
from jax.experimental.pallas import tpu as pltpu

TOKENS = 8192
D_MODEL = 4096
UNITS = 64
TEMP = 0.5

BLOCK_M = 1024


def _router_kernel(x_ref, w_ref, b_ref, o_ref):
    logits = jnp.dot(x_ref[...], w_ref[...], preferred_element_type=jnp.float32)
    logits = logits + b_ref[...]
    o_ref[...] = jax.nn.sigmoid(logits / (TEMP + 1e-08))


def kernel(inputs, proj, logit_bias):
    bias2d = logit_bias.reshape(1, UNITS)
    grid = (TOKENS // BLOCK_M,)
    return pl.pallas_call(
        _router_kernel,
        grid=grid,
        in_specs=[
            pl.BlockSpec((BLOCK_M, D_MODEL), lambda i: (i, 0)),
            pl.BlockSpec((D_MODEL, UNITS), lambda i: (0, 0)),
            pl.BlockSpec((1, UNITS), lambda i: (0, 0)),
        ],
        out_specs=pl.BlockSpec((BLOCK_M, UNITS), lambda i: (i, 0)),
        out_shape=jax.ShapeDtypeStruct((TOKENS, UNITS), jnp.float32),
        compiler_params=pltpu.CompilerParams(
            dimension_semantics=("arbitrary",),
        ),
    )(inputs, proj, bias2d)


# speedup vs baseline: 1.1714x; 1.0301x over previous
import jax
import jax.numpy as jnp
from jax.experimental import pallas as pl
from jax.experimental.pallas import tpu as pltpu

TOKENS = 8192
D_MODEL = 4096
UNITS = 64
TEMP = 0.5

BLOCK_M = 512


def _router_kernel(x_ref, w_ref, b_ref, o_ref):
    logits = jnp.dot(x_ref[...], w_ref[...], preferred_element_type=jnp.float32)
    logits = logits + b_ref[...]
    o_ref[...] = jax.nn.sigmoid(logits / (TEMP + 1e-08))


def kernel(inputs, proj, logit_bias):
    bias2d = logit_bias.reshape(1, UNITS)
    grid = (TOKENS // BLOCK_M,)
    return pl.pallas_call(
        _router_kernel,
        grid=grid,
        in_specs=[
            pl.BlockSpec((BLOCK_M, D_MODEL), lambda i: (i, 0)),
            pl.BlockSpec((D_MODEL, UNITS), lambda i: (0, 0)),
            pl.BlockSpec((1, UNITS), lambda i: (0, 0)),
        ],
        out_specs=pl.BlockSpec((BLOCK_M, UNITS), lambda i: (i, 0)),
        out_shape=jax.ShapeDtypeStruct((TOKENS, UNITS), jnp.float32),
        compiler_params=pltpu.CompilerParams(
            dimension_semantics=("arbitrary",),
        ),
    )(inputs, proj, bias2d)


# speedup vs baseline: 1.1795x; 1.0069x over previous
import jax
import jax.numpy as jnp
from jax.experimental import pallas as pl
from jax.experimental.pallas import tpu as pltpu

TOKENS = 8192
D_MODEL = 4096
UNITS = 64
TEMP = 0.5

BLOCK_M = 512


def _router_kernel(x_ref, w_ref, b_ref, o_ref):
    logits = jnp.dot(x_ref[...], w_ref[...], preferred_element_type=jnp.float32)
    logits = logits + b_ref[...]
    o_ref[...] = jax.nn.sigmoid(logits / (TEMP + 1e-08))


def kernel(inputs, proj, logit_bias):
    bias2d = logit_bias.reshape(1, UNITS)
    grid = (TOKENS // BLOCK_M,)
    return pl.pallas_call(
        _router_kernel,
        grid=grid,
        in_specs=[
            pl.BlockSpec((BLOCK_M, D_MODEL), lambda i: (i, 0)),
            pl.BlockSpec((D_MODEL, UNITS), lambda i: (0, 0)),
            pl.BlockSpec((1, UNITS), lambda i: (0, 0)),
        ],
        out_specs=pl.BlockSpec((BLOCK_M, UNITS), lambda i: (i, 0)),
        out_shape=jax.ShapeDtypeStruct((TOKENS, UNITS), jnp.float32),
        compiler_params=pltpu.CompilerParams(
            dimension_semantics=("parallel",),
        ),
    )(inputs, proj, bias2d)
